# speedup vs baseline: 2.2758x; 2.2758x over previous
import jax
import jax.numpy as jnp
from jax.experimental import pallas as pl
from jax.experimental.pallas import tpu as pltpu

S = 28
N = 90
M = 64 * 28 * 28
CELL_BLOCKS = 49


def _body(p_ref, t_ref, out_ref):
    i = pl.program_id(0)
    p = p_ref[...]
    t = t_ref[...]

    conf = t[4]
    cm = (conf > 0).astype(jnp.float32)

    tx, ty, tw, th = t[0], t[1], t[2], t[3]
    tltx = tx / S - 0.5 * tw
    trbx = tx / S + 0.5 * tw
    tlty = ty / S - 0.5 * th
    trby = ty / S + 0.5 * th
    area_t = (trbx - tltx) * (trby - tlty)

    ious = []
    for b in (0, 1):
        px, py, pw, ph = p[5 * b], p[5 * b + 1], p[5 * b + 2], p[5 * b + 3]
        pltx = px / S - 0.5 * pw
        prbx = px / S + 0.5 * pw
        plty = py / S - 0.5 * ph
        prby = py / S + 0.5 * ph
        whx = jnp.maximum(jnp.minimum(prbx, trbx) - jnp.maximum(pltx, tltx), 0.0)
        why = jnp.maximum(jnp.minimum(prby, trby) - jnp.maximum(plty, tlty), 0.0)
        inter = whx * why
        area_p = (prbx - pltx) * (prby - plty)
        union = area_p + area_t - inter
        ious.append(inter / union)

    sel = ious[1] > ious[0]
    max_iou = jnp.maximum(ious[0], ious[1])

    def pick(f0, f1):
        return jnp.where(sel, f1, f0)

    spx = pick(p[0], p[5])
    spy = pick(p[1], p[6])
    spw = pick(p[2], p[7])
    sph = pick(p[3], p[8])
    spc = pick(p[4], p[9])
    stx = pick(t[0], t[5])
    sty = pick(t[1], t[6])
    stw = pick(t[2], t[7])
    sth = pick(t[3], t[8])

    l_xy = (spx - stx) ** 2 + (spy - sty) ** 2
    l_wh = (jnp.sqrt(spw) - jnp.sqrt(stw)) ** 2 + (jnp.sqrt(sph) - jnp.sqrt(sth)) ** 2
    l_obj = (spc - max_iou) ** 2

    pc = p[10:90]
    tc = t[10:90]
    bce = -(tc * jnp.log(pc) + (1.0 - tc) * jnp.log(1.0 - pc))
    l_cls = jnp.sum(bce, axis=0)

    qp = p[4]
    q = t[4]
    alpha = (1.0 - q) / (1.0 - qp)
    l_df = alpha * (qp - q) * jnp.log(qp) + (q - qp) * jnp.log(1.0 - qp)

    per_cell = cm * (l_xy + l_wh + l_obj + l_cls) + l_df
    tot = jnp.sum(per_cell)

    @pl.when(i == 0)
    def _():
        out_ref[0, 0] = 0.0

    out_ref[0, 0] += tot


def kernel(pred_tensor, target_tensor):
    p = pred_tensor.reshape(M, N).T.reshape(N, CELL_BLOCKS * 8, 128)
    t = target_tensor.reshape(M, N).T.reshape(N, CELL_BLOCKS * 8, 128)
    out = pl.pallas_call(
        _body,
        grid=(CELL_BLOCKS,),
        in_specs=[
            pl.BlockSpec((N, 8, 128), lambda i: (0, i, 0)),
            pl.BlockSpec((N, 8, 128), lambda i: (0, i, 0)),
        ],
        out_specs=pl.BlockSpec(memory_space=pltpu.SMEM),
        out_shape=jax.ShapeDtypeStruct((1, 1), jnp.float32),
    )(p, t)
    return out[0, 0]


# speedup vs baseline: 2.6644x; 1.1707x over previous
import functools

import jax
import jax.numpy as jnp
from jax import lax
from jax.experimental import pallas as pl
from jax.experimental.pallas import tpu as pltpu
from jax.experimental.pallas import tpu_sc as plsc

N = 90
M = 64 * 28 * 28
NW = 32
RPW = M // NW
CH = 112
NCHUNK = RPW // CH
L = 16
INV_S = 1.0 / 28.0

LN2 = 0.6931471805599453
_LOG_COEFFS = (
    7.0376836292e-2, -1.1514610310e-1, 1.1676998740e-1, -1.2420140846e-1,
    1.4249322787e-1, -1.6668057665e-1, 2.0000714765e-1, -2.4999993993e-1,
    3.3333331174e-1,
)


def _plog(x):
    bits = lax.bitcast_convert_type(x, jnp.int32)
    e = ((bits >> 23) & 0xFF) - 126
    m = lax.bitcast_convert_type((bits & 0x007FFFFF) | 0x3F000000, jnp.float32)
    small = m < 0.7071067811865476
    m = jnp.where(small, m + m, m)
    e = jnp.where(small, e - 1, e)
    y = m - 1.0
    z = y * y
    r = jnp.zeros(x.shape, jnp.float32) + _LOG_COEFFS[0]
    for c in _LOG_COEFFS[1:]:
        r = r * y + c
    r = y * z * r - 0.5 * z + y
    return r + e.astype(jnp.float32) * LN2


def _psqrt(x):
    return jnp.exp(0.5 * _plog(x))


def _sc_body(pred_hbm, tgt_hbm, out_hbm, pred_v, tgt_v, idx_v, res_v):
    c = lax.axis_index("c")
    s = lax.axis_index("s")
    wid = s * 2 + c
    base = wid * RPW
    lane = lax.iota(jnp.int32, L)

    def col(f):
        return jnp.zeros((L,), jnp.int32) + f

    def chunk_body(cc, acc):
        row0 = base + cc * CH
        pltpu.sync_copy(pred_hbm.at[pl.ds(row0 * N, CH * N)], pred_v)
        pltpu.sync_copy(tgt_hbm.at[pl.ds(row0 * N, CH * N)], tgt_v)

        k_vec = jnp.zeros((L,), jnp.int32)
        for g in range(CH // L):
            lr = lane + g * L
            lr90 = lr * N + 4
            confv = plsc.load_gather(tgt_v, [lr90])
            qp = plsc.load_gather(pred_v, [lr90])
            alpha = (1.0 - confv) / (1.0 - qp)
            df = (alpha * (qp - confv) * _plog(qp)
                  + (confv - qp) * _plog(1.0 - qp))
            acc = acc + df
            m = confv > 0.0
            pos = jnp.cumsum(m.astype(jnp.int32))
            dst = jnp.where(m, k_vec + pos - 1, CH + 15)
            plsc.store_scatter(idx_v, [dst], lr, mask=m)
            k_vec = k_vec + plsc.all_reduce_population_count(m)

        k_s = jnp.max(k_vec)
        ngroups = (k_s + (L - 1)) // L

        def mgroup(g, acc2):
            pos16 = lane + g * L
            valid = pos16 < k_vec
            r = plsc.load_gather(idx_v, [pos16])
            r = jnp.where(valid, r, 0)
            r90 = r * N

            def pg(f):
                return plsc.load_gather(pred_v, [r90 + f])

            def tg(f):
                return plsc.load_gather(tgt_v, [r90 + f])

            px0, py0, pw0, ph0, pc0 = pg(0), pg(1), pg(2), pg(3), pg(4)
            px1, py1, pw1, ph1, pc1 = pg(5), pg(6), pg(7), pg(8), pg(9)
            tx, ty, tw, th = tg(0), tg(1), tg(2), tg(3)
            tx1, ty1, tw1, th1 = tg(5), tg(6), tg(7), tg(8)

            tltx = tx * INV_S - 0.5 * tw
            trbx = tx * INV_S + 0.5 * tw
            tlty = ty * INV_S - 0.5 * th
            trby = ty * INV_S + 0.5 * th
            area_t = (trbx - tltx) * (trby - tlty)

            def iou(px, py, pw, ph):
                pltx = px * INV_S - 0.5 * pw
                prbx = px * INV_S + 0.5 * pw
                plty = py * INV_S - 0.5 * ph
                prby = py * INV_S + 0.5 * ph
                whx = jnp.maximum(
                    jnp.minimum(prbx, trbx) - jnp.maximum(pltx, tltx), 0.0)
                why = jnp.maximum(
                    jnp.minimum(prby, trby) - jnp.maximum(plty, tlty), 0.0)
                inter = whx * why
                area_p = (prbx - pltx) * (prby - plty)
                return inter / (area_p + area_t - inter)

            i0 = iou(px0, py0, pw0, ph0)
            i1 = iou(px1, py1, pw1, ph1)
            selb = i1 > i0
            mx = jnp.maximum(i0, i1)

            def pick(a, b):
                return jnp.where(selb, b, a)

            spx, spy = pick(px0, px1), pick(py0, py1)
            spw, sph, spc = pick(pw0, pw1), pick(ph0, ph1), pick(pc0, pc1)
            stx, sty = pick(tx, tx1), pick(ty, ty1)
            stw, sth = pick(tw, tw1), pick(th, th1)

            dx, dy, dc = spx - stx, spy - sty, spc - mx
            tot = dx * dx + dy * dy + dc * dc
            tot = tot + spw + stw - 2.0 * _psqrt(spw * stw)
            tot = tot + sph + sth - 2.0 * _psqrt(sph * sth)

            def cls_chunk(j, bce):
                cb = 10 + j * L
                for u in range(L):
                    fc = r90 + (cb + u)
                    pcv = plsc.load_gather(pred_v, [fc])
                    tcv = plsc.load_gather(tgt_v, [fc])
                    bce = bce - (tcv * _plog(pcv)
                                 + (1.0 - tcv) * _plog(1.0 - pcv))
                return bce

            bce = lax.fori_loop(0, (N - 10) // L, cls_chunk,
                                jnp.zeros((L,), jnp.float32))
            tot = tot + bce
            return acc2 + jnp.where(valid, tot, 0.0)

        return lax.fori_loop(0, ngroups, mgroup, acc)

    acc = lax.fori_loop(0, NCHUNK, chunk_body, jnp.zeros((L,), jnp.float32))
    res_v[...] = acc
    pltpu.sync_copy(res_v, out_hbm.at[wid])


_MESH = plsc.VectorSubcoreMesh(core_axis_name="c", subcore_axis_name="s")

_sc_call = functools.partial(
    pl.kernel,
    out_type=jax.ShapeDtypeStruct((NW, L), jnp.float32),
    mesh=_MESH,
    compiler_params=pltpu.CompilerParams(needs_layout_passes=False),
    scratch_types=[
        pltpu.VMEM((CH * N,), jnp.float32),
        pltpu.VMEM((CH * N,), jnp.float32),
        pltpu.VMEM((CH + L, ), jnp.int32),
        pltpu.VMEM((L,), jnp.float32),
    ],
)(_sc_body)


def kernel(pred_tensor, target_tensor):
    p = pred_tensor.reshape(M * N)
    t = target_tensor.reshape(M * N)
    parts = _sc_call(p, t)
    return jnp.sum(parts)
